# Initial kernel scaffold; baseline (speedup 1.0000x reference)
import jax
import jax.numpy as jnp
from jax.experimental import pallas as pl


def kernel(Xs, support_indices, L_values, node_input_features, self_activation, W_neigh_0, W_self_0, w_gate_0, W_neigh_1, W_self_1, w_gate_1):
    raise NotImplementedError("write your pallas kernel here")


# speedup vs baseline: 17.3653x; 17.3653x over previous
import functools

import jax
import jax.numpy as jnp
from jax import lax
from jax.experimental import pallas as pl
from jax.experimental.pallas import tpu as pltpu
from jax.experimental.pallas import tpu_sc as plsc

_N = 10000
_E = 160000
_D = 128
_B = 2

_NCORE = 2
_NSUB = 16
_EPT = _E // _NSUB
_G = 80
_NG = 5
_CHUNK = _G * _NG
_NCHUNK = _EPT // _CHUNK
_HN = 5000
_NPASS = 2
_ACC = _HN + 8
_RPT = 312
_RL0 = _ACC - 15 * _RPT


def _zero_f32(ref, nelems):
    def body(i, _):
        ref[pl.ds(i * 16, 16)] = jnp.zeros((16,), jnp.float32)
        return 0
    lax.fori_loop(0, nelems // 16, body, 0, unroll=True)


def _stage_edges(src_hbm, dst_hbm, lv_hbm, base, src_i, dst_i, l_v):
    for j in range(_NG):
        pltpu.sync_copy(src_hbm.at[pl.ds(base + _G * j, _G)], src_i.at[j])
        pltpu.sync_copy(dst_hbm.at[pl.ds(base + _G * j, _G)], dst_i.at[j])
        pltpu.sync_copy(lv_hbm.at[pl.ds(base + _G * j, _G)], l_v.at[j])


def _sc_layer0(inf_hbm, st_hbm, g_hbm, src_hbm, dst_hbm, lv_hbm,
               accinf_hbm, accst_hbm,
               st_v, g_v, src_i, dst_i, l_v, dstl_i, a_v, vals_v,
               rows_v, z2d_v, z1d_v, acc_s, accs_s, gsem):
    c = lax.axis_index("c")
    s = lax.axis_index("s")

    pltpu.sync_copy(st_hbm.at[pl.ds(c * _N, _N)], st_v)
    pltpu.sync_copy(g_hbm, g_v)

    _zero_f32(z1d_v, 1008)
    def zb(i, _):
        z2d_v[i // 8, pl.ds(16 * (i % 8), 16)] = jnp.zeros((16,), jnp.float32)
        return 0
    lax.fori_loop(0, 8 * 8, zb, 0, unroll=True)

    for p in range(_NPASS):
        nz = jnp.where(s < 15, _RPT // 8, _RL0 // 8)
        def zrow(r, _):
            pltpu.sync_copy(z2d_v, acc_s.at[pl.ds(s * _RPT + 8 * r, 8)])
            return 0
        lax.fori_loop(0, nz, zrow, 0)
        if p == 0:
            @pl.when(s < 10)
            def _():
                pltpu.sync_copy(z1d_v.at[pl.ds(0, 1000)],
                                accs_s.at[pl.ds(s * 1000, 1000)])
        plsc.subcore_barrier()

        lo = jnp.full((16,), p * _HN, jnp.int32)

        def chunk(ch, _):
            base = s * _EPT + ch * _CHUNK
            _stage_edges(src_hbm, dst_hbm, lv_hbm, base, src_i, dst_i, l_v)
            cps = [pltpu.async_copy(
                       inf_hbm.at[src_i.at[j]],
                       rows_v.at[pl.ds(_G * j, _G)], gsem)
                   for j in range(_NG)]
            for j in range(_NG):
                for g in range(_NG):
                    idx = src_i[j, pl.ds(16 * g, 16)]
                    d = dst_i[j, pl.ds(16 * g, 16)]
                    a = l_v[j, pl.ds(16 * g, 16)] * plsc.load_gather(st_v, [idx])
                    a_v[j, pl.ds(16 * g, 16)] = a
                    dl = d - lo
                    ok = (dl >= 0) & (dl < _HN)
                    dstl_i[j, pl.ds(16 * g, 16)] = jnp.where(
                        ok, dl, jnp.full((16,), _HN, jnp.int32))
                    if p == 0:
                        vals_v[j, pl.ds(16 * g, 16)] = (
                            a * plsc.load_gather(g_v, [idx]))
            if p == 0:
                for j in range(_NG):
                    pltpu.sync_copy(vals_v.at[j],
                                    accs_s.at[dst_i.at[j]], add=True)
            for cp in cps:
                cp.wait()
            def scale(r, _):
                av = plsc.load_gather(
                    a_v, [jnp.full((16,), r // _G, jnp.int32),
                          jnp.full((16,), r % _G, jnp.int32)])
                for q in range(8):
                    rows_v[r, pl.ds(16 * q, 16)] = (
                        rows_v[r, pl.ds(16 * q, 16)] * av)
                return 0
            lax.fori_loop(0, _CHUNK, scale, 0)
            for j in range(_NG):
                pltpu.sync_copy(rows_v.at[pl.ds(_G * j, _G)],
                                acc_s.at[dstl_i.at[j]], add=True)
            return 0

        lax.fori_loop(0, _NCHUNK, chunk, 0)
        plsc.subcore_barrier()

        valid = min(_HN, _N - p * _HN)
        def out_rows(off, cnt):
            pltpu.sync_copy(acc_s.at[pl.ds(off, cnt)], rows_v.at[pl.ds(0, cnt)])
            pltpu.sync_copy(rows_v.at[pl.ds(0, cnt)],
                            accinf_hbm.at[pl.ds(c * _N + p * _HN + off, cnt)])
        @pl.when(s < 15)
        def _():
            out_rows(s * _RPT, _RPT)
        @pl.when(s == 15)
        def _():
            out_rows(15 * _RPT, valid - 15 * _RPT)
        if p == 0:
            @pl.when(s < 10)
            def _():
                pltpu.sync_copy(accs_s.at[pl.ds(s * 1000, 1000)],
                                z1d_v.at[pl.ds(0, 1000)])
                pltpu.sync_copy(z1d_v.at[pl.ds(0, 1000)],
                                accst_hbm.at[pl.ds(c * _N + s * 1000, 1000)])
        if p < _NPASS - 1:
            plsc.subcore_barrier()


def _sc_layer1(st_hbm, g_hbm, src_hbm, dst_hbm, lv_hbm, accst_hbm,
               st_v, g_v, src_i, dst_i, l_v, a_v, vals_v, z1d_v, accs_s):
    c = lax.axis_index("c")
    s = lax.axis_index("s")
    pltpu.sync_copy(st_hbm.at[pl.ds(c * _N, _N)], st_v)
    pltpu.sync_copy(g_hbm.at[pl.ds(c * _N, _N)], g_v)
    _zero_f32(z1d_v, 1008)
    @pl.when(s < 10)
    def _():
        pltpu.sync_copy(z1d_v.at[pl.ds(0, 1000)],
                        accs_s.at[pl.ds(s * 1000, 1000)])
    plsc.subcore_barrier()

    def chunk(ch, _):
        base = s * _EPT + ch * _CHUNK
        _stage_edges(src_hbm, dst_hbm, lv_hbm, base, src_i, dst_i, l_v)
        for j in range(_NG):
            for g in range(_NG):
                idx = src_i[j, pl.ds(16 * g, 16)]
                a = l_v[j, pl.ds(16 * g, 16)] * plsc.load_gather(st_v, [idx])
                vals_v[j, pl.ds(16 * g, 16)] = (
                    a * plsc.load_gather(g_v, [idx]))
        for j in range(_NG):
            pltpu.sync_copy(vals_v.at[j], accs_s.at[dst_i.at[j]], add=True)
        return 0

    lax.fori_loop(0, _NCHUNK, chunk, 0)
    plsc.subcore_barrier()
    @pl.when(s < 10)
    def _():
        pltpu.sync_copy(accs_s.at[pl.ds(s * 1000, 1000)],
                        z1d_v.at[pl.ds(0, 1000)])
        pltpu.sync_copy(z1d_v.at[pl.ds(0, 1000)],
                        accst_hbm.at[pl.ds(c * _N + s * 1000, 1000)])


def _tc_pre(nif_ref, xs_ref, sa_ref, wg0_ref, inf0_ref, st0_ref, gate0_ref):
    nif = nif_ref[...]
    m = jnp.mean(nif, axis=0, keepdims=True)
    xc = nif - m
    v = jnp.mean(xc * xc, axis=0, keepdims=True)
    inf0 = xc * lax.rsqrt(v + 1e-5)
    inf0_ref[...] = inf0
    gate0_ref[...] = jax.nn.sigmoid(
        jnp.dot(inf0, wg0_ref[...], preferred_element_type=jnp.float32))
    xs = xs_ref[...]
    st0_ref[...] = (xs + sa_ref[...]) * (1.0 - xs) + xs


def _tc_mid(accinf_ref, inf0_ref, accst_ref, st0_ref, wn_ref, ws_ref,
            wg1_ref, st1_ref, g1_ref):
    st1_ref[...] = st0_ref[...] + accst_ref[...]
    shared = jnp.dot(inf0_ref[...], ws_ref[...],
                     preferred_element_type=jnp.float32)
    m1 = jnp.dot(accinf_ref[...], wn_ref[...],
                 preferred_element_type=jnp.float32)
    inf1 = jax.nn.relu(m1 + jnp.concatenate([shared, shared], axis=0))
    g1_ref[...] = jax.nn.sigmoid(
        jnp.dot(inf1, wg1_ref[...], preferred_element_type=jnp.float32))


def _tc_post(st1_ref, acc2_ref, xs_ref, out_ref):
    xs = xs_ref[...]
    out = jnp.tanh(st1_ref[...] + acc2_ref[...])
    out_ref[...] = out * (1.0 - xs) + xs


def _sds(shape):
    return jax.ShapeDtypeStruct(shape, jnp.float32)


_sc1_scratch = [
    pltpu.VMEM((_N,), jnp.float32),
    pltpu.VMEM((_N,), jnp.float32),
    pltpu.VMEM((_NG, _G), jnp.int32),
    pltpu.VMEM((_NG, _G), jnp.int32),
    pltpu.VMEM((_NG, _G), jnp.float32),
    pltpu.VMEM((_NG, _G), jnp.int32),
    pltpu.VMEM((_NG, _G), jnp.float32),
    pltpu.VMEM((_NG, _G), jnp.float32),
    pltpu.VMEM((_CHUNK, _D), jnp.float32),
    pltpu.VMEM((8, _D), jnp.float32),
    pltpu.VMEM((1008,), jnp.float32),
    pltpu.VMEM_SHARED((_ACC, _D), jnp.float32),
    pltpu.VMEM_SHARED((_N,), jnp.float32),
    pltpu.SemaphoreType.DMA,
]

_sc2_scratch = [
    pltpu.VMEM((_N,), jnp.float32),
    pltpu.VMEM((_N,), jnp.float32),
    pltpu.VMEM((_NG, _G), jnp.int32),
    pltpu.VMEM((_NG, _G), jnp.int32),
    pltpu.VMEM((_NG, _G), jnp.float32),
    pltpu.VMEM((_NG, _G), jnp.float32),
    pltpu.VMEM((_NG, _G), jnp.float32),
    pltpu.VMEM((1008,), jnp.float32),
    pltpu.VMEM_SHARED((_N,), jnp.float32),
]

_sc_mesh = plsc.VectorSubcoreMesh(
    core_axis_name="c", subcore_axis_name="s",
    num_cores=_NCORE, num_subcores=_NSUB)

_sc_params = pltpu.CompilerParams(needs_layout_passes=False)

_sc1_call = functools.partial(
    pl.kernel,
    out_type=(_sds((_B * _N, _D)), _sds((_B * _N,))),
    mesh=_sc_mesh, scratch_types=_sc1_scratch,
    compiler_params=_sc_params)(_sc_layer0)

_sc2_call = functools.partial(
    pl.kernel, out_type=_sds((_B * _N,)),
    mesh=_sc_mesh, scratch_types=_sc2_scratch,
    compiler_params=_sc_params)(_sc_layer1)

_tc_pre_call = pl.pallas_call(
    _tc_pre,
    out_shape=(_sds((_N, _D)), _sds((_N, _B)), _sds((_N, 1))))

_tc_mid_call = pl.pallas_call(
    _tc_mid,
    out_shape=(_sds((_N, _B)), _sds((_B * _N, 1))))

_tc_post_call = pl.pallas_call(
    _tc_post,
    out_shape=_sds((_N, _B)))


def kernel(Xs, support_indices, L_values, node_input_features,
           self_activation, W_neigh_0, W_self_0, w_gate_0,
           W_neigh_1, W_self_1, w_gate_1):
    dst = support_indices[:, 0].astype(jnp.int32)
    src = support_indices[:, 1].astype(jnp.int32)
    xs_nb = jnp.transpose(Xs[:, :, 0])

    inf0, st0, gate0 = _tc_pre_call(
        node_input_features, xs_nb, self_activation, w_gate_0)

    accinf, accst = _sc1_call(
        inf0, jnp.reshape(jnp.transpose(st0), (_B * _N,)),
        gate0[:, 0], src, dst, L_values)

    st1, g1 = _tc_mid_call(
        accinf, inf0,
        jnp.transpose(jnp.reshape(accst, (_B, _N))), st0,
        W_neigh_0, W_self_0, w_gate_1)

    acc2 = _sc2_call(jnp.reshape(jnp.transpose(st1), (_B * _N,)), g1[:, 0],
                     src, dst, L_values)

    out = _tc_post_call(st1, jnp.transpose(jnp.reshape(acc2, (_B, _N))), xs_nb)
    return jnp.transpose(out)[:, :, None]


# speedup vs baseline: 27.0673x; 1.5587x over previous
import functools

import jax
import jax.numpy as jnp
from jax import lax
from jax.experimental import pallas as pl
from jax.experimental.pallas import tpu as pltpu
from jax.experimental.pallas import tpu_sc as plsc

_N = 10000
_E = 160000
_D = 128
_B = 2

_NCORE = 2
_NSUB = 16
_EPT = _E // _NSUB
_G = 80
_NG = 5
_CHUNK = _G * _NG
_NCHUNK = _EPT // _CHUNK
_HN = 5000
_NPASS = 2
_ACC = _HN + 8
_RPT = 312
_RL0 = _ACC - 15 * _RPT


def _zero_f32(ref, nelems):
    def body(i, _):
        ref[pl.ds(i * 16, 16)] = jnp.zeros((16,), jnp.float32)
        return 0
    lax.fori_loop(0, nelems // 16, body, 0, unroll=True)


def _stage_edges(src_hbm, dst_hbm, lv_hbm, base, src_i, dst_i, l_v):
    for j in range(_NG):
        pltpu.sync_copy(src_hbm.at[pl.ds(base + _G * j, _G)], src_i.at[j])
        pltpu.sync_copy(dst_hbm.at[pl.ds(base + _G * j, _G)], dst_i.at[j])
        pltpu.sync_copy(lv_hbm.at[pl.ds(base + _G * j, _G)], l_v.at[j])


def _sc_layer0(inf_hbm, st_hbm, g_hbm, src_hbm, dst_hbm, lv_hbm,
               accinf_hbm, accst_hbm,
               st_v, g_v, lv_v, src1d, dst1d, dsto_i, dstl_i, a_v, vals_v,
               rows_v, z2d_v, z1d_v, acc_s, accs_s, gsem):
    c = lax.axis_index("c")
    s = lax.axis_index("s")

    pltpu.sync_copy(st_hbm.at[pl.ds(c * _N, _N)], st_v)
    pltpu.sync_copy(g_hbm, g_v)
    pltpu.sync_copy(lv_hbm.at[pl.ds(s * _EPT, _EPT)], lv_v)

    _zero_f32(z1d_v, 1008)
    def zb(i, _):
        z2d_v[i // 8, pl.ds(16 * (i % 8), 16)] = jnp.zeros((16,), jnp.float32)
        return 0
    lax.fori_loop(0, 8 * 8, zb, 0, unroll=True)

    for p in range(_NPASS):
        nz = jnp.where(s < 15, _RPT // 8, _RL0 // 8)
        def zrow(r, _):
            pltpu.sync_copy(z2d_v, acc_s.at[pl.ds(s * _RPT + 8 * r, 8)])
            return 0
        lax.fori_loop(0, nz, zrow, 0)
        if p == 0:
            @pl.when(s < 10)
            def _():
                pltpu.sync_copy(z1d_v.at[pl.ds(0, 1000)],
                                accs_s.at[pl.ds(s * 1000, 1000)])
        plsc.subcore_barrier()

        lo = jnp.full((16,), p * _HN, jnp.int32)

        def chunk(ch, _):
            base = s * _EPT + ch * _CHUNK
            pltpu.sync_copy(src_hbm.at[pl.ds(base, _CHUNK)], src1d)
            pltpu.sync_copy(dst_hbm.at[pl.ds(base, _CHUNK)], dst1d)
            cps = [pltpu.async_copy(
                       inf_hbm.at[src1d.at[pl.ds(_G * j, _G)]],
                       rows_v.at[pl.ds(_G * j, _G)], gsem)
                   for j in range(_NG)]
            for j in range(_NG):
                for g in range(_NG):
                    e = _G * j + 16 * g
                    idx = src1d[pl.ds(e, 16)]
                    d = dst1d[pl.ds(e, 16)]
                    a = (lv_v[pl.ds(ch * _CHUNK + e, 16)]
                         * plsc.load_gather(st_v, [idx]))
                    a_v[j, pl.ds(16 * g, 16)] = a
                    dl = d - lo
                    ok = (dl >= 0) & (dl < _HN)
                    dstl_i[j, pl.ds(16 * g, 16)] = jnp.where(
                        ok, dl, jnp.full((16,), _HN, jnp.int32))
                    if p == 0:
                        dsto_i[j, pl.ds(16 * g, 16)] = d
                        vals_v[j, pl.ds(16 * g, 16)] = (
                            a * plsc.load_gather(g_v, [idx]))
            if p == 0:
                for j in range(_NG):
                    pltpu.sync_copy(vals_v.at[j],
                                    accs_s.at[dsto_i.at[j]], add=True)
            for cp in cps:
                cp.wait()
            def scale(r, _):
                av = plsc.load_gather(
                    a_v, [jnp.full((16,), r // _G, jnp.int32),
                          jnp.full((16,), r % _G, jnp.int32)])
                for q in range(8):
                    rows_v[r, pl.ds(16 * q, 16)] = (
                        rows_v[r, pl.ds(16 * q, 16)] * av)
                return 0
            lax.fori_loop(0, _CHUNK, scale, 0)
            for j in range(_NG):
                pltpu.sync_copy(rows_v.at[pl.ds(_G * j, _G)],
                                acc_s.at[dstl_i.at[j]], add=True)
            return 0

        lax.fori_loop(0, _NCHUNK, chunk, 0)
        plsc.subcore_barrier()

        valid = min(_HN, _N - p * _HN)
        def out_rows(off, cnt):
            pltpu.sync_copy(acc_s.at[pl.ds(off, cnt)], rows_v.at[pl.ds(0, cnt)])
            pltpu.sync_copy(rows_v.at[pl.ds(0, cnt)],
                            accinf_hbm.at[pl.ds(c * _N + p * _HN + off, cnt)])
        @pl.when(s < 15)
        def _():
            out_rows(s * _RPT, _RPT)
        @pl.when(s == 15)
        def _():
            out_rows(15 * _RPT, valid - 15 * _RPT)
        if p == 0:
            @pl.when(s < 10)
            def _():
                pltpu.sync_copy(accs_s.at[pl.ds(s * 1000, 1000)],
                                z1d_v.at[pl.ds(0, 1000)])
                pltpu.sync_copy(z1d_v.at[pl.ds(0, 1000)],
                                accst_hbm.at[pl.ds(c * _N + s * 1000, 1000)])
        if p < _NPASS - 1:
            plsc.subcore_barrier()


def _sc_layer1(st_hbm, g_hbm, src_hbm, dst_hbm, lv_hbm, accst_hbm,
               st_v, g_v, lv_v, src1d, dst1d, dsto_i, a_v, vals_v, z1d_v,
               accs_s):
    c = lax.axis_index("c")
    s = lax.axis_index("s")
    pltpu.sync_copy(st_hbm.at[pl.ds(c * _N, _N)], st_v)
    pltpu.sync_copy(g_hbm.at[pl.ds(c * _N, _N)], g_v)
    pltpu.sync_copy(lv_hbm.at[pl.ds(s * _EPT, _EPT)], lv_v)
    _zero_f32(z1d_v, 1008)
    @pl.when(s < 10)
    def _():
        pltpu.sync_copy(z1d_v.at[pl.ds(0, 1000)],
                        accs_s.at[pl.ds(s * 1000, 1000)])
    plsc.subcore_barrier()

    def chunk(ch, _):
        base = s * _EPT + ch * _CHUNK
        pltpu.sync_copy(src_hbm.at[pl.ds(base, _CHUNK)], src1d)
        pltpu.sync_copy(dst_hbm.at[pl.ds(base, _CHUNK)], dst1d)
        for j in range(_NG):
            for g in range(_NG):
                e = _G * j + 16 * g
                idx = src1d[pl.ds(e, 16)]
                a = (lv_v[pl.ds(ch * _CHUNK + e, 16)]
                     * plsc.load_gather(st_v, [idx]))
                vals_v[j, pl.ds(16 * g, 16)] = (
                    a * plsc.load_gather(g_v, [idx]))
                dsto_i[j, pl.ds(16 * g, 16)] = dst1d[pl.ds(e, 16)]
        for j in range(_NG):
            pltpu.sync_copy(vals_v.at[j], accs_s.at[dsto_i.at[j]], add=True)
        return 0

    lax.fori_loop(0, _NCHUNK, chunk, 0)
    plsc.subcore_barrier()
    @pl.when(s < 10)
    def _():
        pltpu.sync_copy(accs_s.at[pl.ds(s * 1000, 1000)],
                        z1d_v.at[pl.ds(0, 1000)])
        pltpu.sync_copy(z1d_v.at[pl.ds(0, 1000)],
                        accst_hbm.at[pl.ds(c * _N + s * 1000, 1000)])


def _tc_pre(nif_ref, xs_ref, sa_ref, wg0_ref, inf0_ref, st0_ref, gate0_ref):
    nif = nif_ref[...]
    m = jnp.mean(nif, axis=0, keepdims=True)
    xc = nif - m
    v = jnp.mean(xc * xc, axis=0, keepdims=True)
    inf0 = xc * lax.rsqrt(v + 1e-5)
    inf0_ref[...] = inf0
    gate0_ref[...] = jax.nn.sigmoid(
        jnp.dot(inf0, wg0_ref[...], preferred_element_type=jnp.float32))
    xs = xs_ref[...]
    st0_ref[...] = (xs + sa_ref[...]) * (1.0 - xs) + xs


def _tc_mid(accinf_ref, inf0_ref, accst_ref, st0_ref, wn_ref, ws_ref,
            wg1_ref, st1_ref, g1_ref):
    st1_ref[...] = st0_ref[...] + accst_ref[...]
    shared = jnp.dot(inf0_ref[...], ws_ref[...],
                     preferred_element_type=jnp.float32)
    m1 = jnp.dot(accinf_ref[...], wn_ref[...],
                 preferred_element_type=jnp.float32)
    inf1 = jax.nn.relu(m1 + jnp.concatenate([shared, shared], axis=0))
    g1_ref[...] = jax.nn.sigmoid(
        jnp.dot(inf1, wg1_ref[...], preferred_element_type=jnp.float32))


def _tc_post(st1_ref, acc2_ref, xs_ref, out_ref):
    xs = xs_ref[...]
    out = jnp.tanh(st1_ref[...] + acc2_ref[...])
    out_ref[...] = out * (1.0 - xs) + xs


def _sds(shape):
    return jax.ShapeDtypeStruct(shape, jnp.float32)


_sc1_scratch = [
    pltpu.VMEM((_N,), jnp.float32),
    pltpu.VMEM((_N,), jnp.float32),
    pltpu.VMEM((_EPT,), jnp.float32),
    pltpu.VMEM((_CHUNK,), jnp.int32),
    pltpu.VMEM((_CHUNK,), jnp.int32),
    pltpu.VMEM((_NG, _G), jnp.int32),
    pltpu.VMEM((_NG, _G), jnp.int32),
    pltpu.VMEM((_NG, _G), jnp.float32),
    pltpu.VMEM((_NG, _G), jnp.float32),
    pltpu.VMEM((_CHUNK, _D), jnp.float32),
    pltpu.VMEM((8, _D), jnp.float32),
    pltpu.VMEM((1008,), jnp.float32),
    pltpu.VMEM_SHARED((_ACC, _D), jnp.float32),
    pltpu.VMEM_SHARED((_N,), jnp.float32),
    pltpu.SemaphoreType.DMA,
]

_sc2_scratch = [
    pltpu.VMEM((_N,), jnp.float32),
    pltpu.VMEM((_N,), jnp.float32),
    pltpu.VMEM((_EPT,), jnp.float32),
    pltpu.VMEM((_CHUNK,), jnp.int32),
    pltpu.VMEM((_CHUNK,), jnp.int32),
    pltpu.VMEM((_NG, _G), jnp.int32),
    pltpu.VMEM((_NG, _G), jnp.float32),
    pltpu.VMEM((_NG, _G), jnp.float32),
    pltpu.VMEM((1008,), jnp.float32),
    pltpu.VMEM_SHARED((_N,), jnp.float32),
]

_sc_mesh = plsc.VectorSubcoreMesh(
    core_axis_name="c", subcore_axis_name="s",
    num_cores=_NCORE, num_subcores=_NSUB)

_sc_params = pltpu.CompilerParams(needs_layout_passes=False)

_sc1_call = functools.partial(
    pl.kernel,
    out_type=(_sds((_B * _N, _D)), _sds((_B * _N,))),
    mesh=_sc_mesh, scratch_types=_sc1_scratch,
    compiler_params=_sc_params)(_sc_layer0)

_sc2_call = functools.partial(
    pl.kernel, out_type=_sds((_B * _N,)),
    mesh=_sc_mesh, scratch_types=_sc2_scratch,
    compiler_params=_sc_params)(_sc_layer1)

_tc_pre_call = pl.pallas_call(
    _tc_pre,
    out_shape=(_sds((_N, _D)), _sds((_N, _B)), _sds((_N, 1))))

_tc_mid_call = pl.pallas_call(
    _tc_mid,
    out_shape=(_sds((_N, _B)), _sds((_B * _N, 1))))

_tc_post_call = pl.pallas_call(
    _tc_post,
    out_shape=_sds((_N, _B)))


def kernel(Xs, support_indices, L_values, node_input_features,
           self_activation, W_neigh_0, W_self_0, w_gate_0,
           W_neigh_1, W_self_1, w_gate_1):
    dst = support_indices[:, 0].astype(jnp.int32)
    src = support_indices[:, 1].astype(jnp.int32)
    xs_nb = jnp.transpose(Xs[:, :, 0])

    inf0, st0, gate0 = _tc_pre_call(
        node_input_features, xs_nb, self_activation, w_gate_0)

    accinf, accst = _sc1_call(
        inf0, jnp.reshape(jnp.transpose(st0), (_B * _N,)),
        gate0[:, 0], src, dst, L_values)

    st1, g1 = _tc_mid_call(
        accinf, inf0,
        jnp.transpose(jnp.reshape(accst, (_B, _N))), st0,
        W_neigh_0, W_self_0, w_gate_1)

    acc2 = _sc2_call(jnp.reshape(jnp.transpose(st1), (_B * _N,)), g1[:, 0],
                     src, dst, L_values)

    out = _tc_post_call(st1, jnp.transpose(jnp.reshape(acc2, (_B, _N))), xs_nb)
    return jnp.transpose(out)[:, :, None]
